# Initial kernel scaffold; baseline (speedup 1.0000x reference)
import jax
import jax.numpy as jnp
from jax.experimental import pallas as pl


def kernel(x, Wr, gate, up, down):
    raise NotImplementedError("write your pallas kernel here")


# speedup vs baseline: 1.1367x; 1.1367x over previous
import functools

import jax
import jax.numpy as jnp
from jax.experimental import pallas as pl
from jax.experimental.pallas import tpu as pltpu


def _router_body(x_ref, wr_ref, logits_ref, w_ref):
    x = x_ref[...]
    wr = wr_ref[...]
    logits = jax.lax.dot_general(
        x, wr, (((1,), (1,)), ((), ())), preferred_element_type=jnp.float32)
    logits_ref[...] = logits
    m = jnp.max(logits, axis=1, keepdims=True)
    unnorm = jnp.exp(logits - m)
    p = unnorm / jnp.sum(unnorm, axis=1, keepdims=True)
    cols = jax.lax.broadcasted_iota(jnp.int32, p.shape, 1)
    i1 = jnp.argmax(p, axis=1)[:, None]
    p_m = jnp.where(cols == i1, -jnp.inf, p)
    i2 = jnp.argmax(p_m, axis=1)[:, None]
    w_ref[...] = jnp.where((cols == i1) | (cols == i2), p, 0.0)


def _ffn_body(nf, x_ref, w_ref, g_ref, u_ref, d_ref, out_ref):
    e = pl.program_id(1)
    f = pl.program_id(2)

    @pl.when((e == 0) & (f == 0))
    def _():
        out_ref[...] = jnp.zeros_like(out_ref)

    xb = x_ref[...].astype(jnp.bfloat16)
    gw = g_ref[0]
    uw = u_ref[0]
    g = jax.lax.dot_general(
        xb, gw, (((1,), (1,)), ((), ())), preferred_element_type=jnp.float32)
    u = jax.lax.dot_general(
        xb, uw, (((1,), (1,)), ((), ())), preferred_element_type=jnp.float32)
    h = (g * jax.nn.sigmoid(g)) * u
    hb = h.astype(jnp.bfloat16)
    dn = d_ref[0]
    y = jax.lax.dot_general(
        hb, dn, (((1,), (1,)), ((), ())), preferred_element_type=jnp.float32)
    cols = jax.lax.broadcasted_iota(jnp.int32, w_ref.shape, 1)
    we = jnp.sum(w_ref[...] * (cols == e).astype(jnp.float32), axis=1,
                 keepdims=True)
    out_ref[...] += we * y


def kernel(x, Wr, gate, up, down):
    b, s, d = x.shape
    e_num, ff, _ = gate.shape
    t = b * s
    xf = x.reshape(t, d)

    bt_r = 1024
    logits, w = pl.pallas_call(
        _router_body,
        grid=(t // bt_r,),
        in_specs=[
            pl.BlockSpec((bt_r, d), lambda i: (i, 0)),
            pl.BlockSpec((e_num, d), lambda i: (0, 0)),
        ],
        out_specs=[
            pl.BlockSpec((bt_r, e_num), lambda i: (i, 0)),
            pl.BlockSpec((bt_r, e_num), lambda i: (i, 0)),
        ],
        out_shape=[
            jax.ShapeDtypeStruct((t, e_num), jnp.float32),
            jax.ShapeDtypeStruct((t, e_num), jnp.float32),
        ],
    )(xf, Wr)

    gate_b = gate.astype(jnp.bfloat16)
    up_b = up.astype(jnp.bfloat16)
    down_b = down.astype(jnp.bfloat16)

    bt = 2048
    fc = 512
    nf = ff // fc
    final = pl.pallas_call(
        functools.partial(_ffn_body, nf),
        grid=(t // bt, e_num, nf),
        in_specs=[
            pl.BlockSpec((bt, d), lambda i, e, f: (i, 0)),
            pl.BlockSpec((bt, e_num), lambda i, e, f: (i, 0)),
            pl.BlockSpec((1, fc, d), lambda i, e, f: (e, f, 0)),
            pl.BlockSpec((1, fc, d), lambda i, e, f: (e, f, 0)),
            pl.BlockSpec((1, d, fc), lambda i, e, f: (e, 0, f)),
        ],
        out_specs=pl.BlockSpec((bt, d), lambda i, e, f: (i, 0)),
        out_shape=jax.ShapeDtypeStruct((t, d), jnp.float32),
        compiler_params=pltpu.CompilerParams(
            dimension_semantics=("arbitrary", "arbitrary", "arbitrary")),
    )(xf, w, gate_b, up_b, down_b)

    return final.reshape(b, s, d), logits


# speedup vs baseline: 2.2785x; 2.0045x over previous
import functools

import jax
import jax.numpy as jnp
from jax import lax
from jax.experimental import pallas as pl
from jax.experimental.pallas import tpu as pltpu
from jax.experimental.pallas import tpu_sc as plsc

T = 8192
D = 1024
E = 8
FF = 4096
KTOP = 2
A = T * KTOP
SQ = 128
BT = 256
L = A + E * BT
NL = L // BT
FC = 512

NC = 2
NS = 16
NW = NC * NS
RPW = L // NW
GC = 32
TPW = T // NW
CT = 32


def _router_body(x_ref, wr_ref, logits_ref, i12_ref, w12_ref):
    x = x_ref[...]
    wr = wr_ref[...]
    logits = jax.lax.dot_general(
        x, wr, (((1,), (1,)), ((), ())), preferred_element_type=jnp.float32)
    logits_ref[...] = logits
    m = jnp.max(logits, axis=1, keepdims=True)
    unnorm = jnp.exp(logits - m)
    p = unnorm / jnp.sum(unnorm, axis=1, keepdims=True)
    cols = jax.lax.broadcasted_iota(jnp.int32, p.shape, 1)
    i1 = jnp.argmax(p, axis=1)[:, None]
    p_m = jnp.where(cols == i1, -jnp.inf, p)
    i2 = jnp.argmax(p_m, axis=1)[:, None]
    w1 = jnp.sum(jnp.where(cols == i1, p, 0.0), axis=1, keepdims=True)
    w2 = jnp.sum(jnp.where(cols == i2, p, 0.0), axis=1, keepdims=True)
    i12_ref[...] = jnp.concatenate([i1, i2], axis=1)
    w12_ref[...] = jnp.concatenate([w1, w2], axis=1)


def _dispatch_body(ti_ref, dest_ref, eid_ref):
    ti = ti_ref[...]
    r_i = jax.lax.broadcasted_iota(jnp.int32, (SQ, SQ), 0)
    c_i = jax.lax.broadcasted_iota(jnp.int32, (SQ, SQ), 1)
    incl = (r_i <= c_i).astype(jnp.float32)
    strict = (c_i < r_i).astype(jnp.float32)

    masks, counts = [], []
    ranks = []
    for e in range(E):
        mf = (ti == e).astype(jnp.float32)
        cum = jax.lax.dot_general(
            mf, incl, (((1,), (0,)), ((), ())),
            preferred_element_type=jnp.float32)
        rowsum = cum[:, SQ - 1:SQ]
        rowpref = jax.lax.dot_general(
            strict, rowsum, (((1,), (0,)), ((), ())),
            preferred_element_type=jnp.float32)
        rank = cum - mf + rowpref
        masks.append(mf)
        ranks.append(rank)
        counts.append(jnp.sum(mf))

    base = 0.0
    dest = jnp.zeros((SQ, SQ), jnp.float32)
    bases = []
    for e in range(E):
        bases.append(base)
        dest = dest + masks[e] * (ranks[e] + base)
        padded = jnp.ceil(counts[e] / BT) * BT
        base = base + padded
    dest_ref[...] = dest.astype(jnp.int32)

    ts = jax.lax.broadcasted_iota(jnp.int32, (1, SQ), 1).astype(
        jnp.float32) * BT
    eid = jnp.zeros((1, SQ), jnp.int32)
    for e in range(1, E):
        eid = eid + (ts >= bases[e]).astype(jnp.int32)
    eid_ref[...] = eid


def _sc_dispatch_body(dest_hbm, tw_hbm, x_hbm, xs_hbm, ws_hbm,
                      tok_v, w_v, dst_v, tw_v, vals_v, idx_v, rows_v,
                      wslice_v, stok_sh, sw_sh, sem):
    cid = lax.axis_index("c")
    sid = lax.axis_index("s")

    @pl.when(sid == 0)
    def _():
        pltpu.sync_copy(dest_hbm, dst_v)
        pltpu.sync_copy(tw_hbm, tw_v)

        def zero_body(i, c):
            tok_v[pl.ds(i * 16, 16)] = jnp.zeros((16,), jnp.int32)
            w_v[pl.ds(i * 16, 16)] = jnp.zeros((16,), jnp.float32)
            return c
        lax.fori_loop(0, L // 16, zero_body, 0)

        def val_body(i, c):
            va = i * 16 + jax.lax.iota(jnp.int32, 16)
            vals_v[pl.ds(i * 16, 16)] = jnp.bitwise_and(va, T - 1)
            return c
        lax.fori_loop(0, A // 16, val_body, 0)

        pltpu.sync_copy(tok_v, stok_sh)
        pltpu.sync_copy(w_v, sw_sh)

        for arr_src, arr_dst in ((vals_v, stok_sh), (tw_v, sw_sh)):
            for g in range(SQ // 16):
                hs = [
                    pltpu.async_copy(
                        arr_src.at[pl.ds((g * 16 + j) * SQ, SQ)],
                        arr_dst.at[dst_v.at[g * 16 + j]],
                        sem)
                    for j in range(16)
                ]
                for h in hs:
                    h.wait()

    plsc.subcore_barrier()

    wid = cid * NS + sid
    base = wid * RPW

    def chunk_body(c, carry):
        off = base + c * GC
        pltpu.sync_copy(stok_sh.at[pl.ds(off, GC)], idx_v)
        pltpu.async_copy(x_hbm.at[idx_v], rows_v, sem).wait()
        pltpu.sync_copy(rows_v, xs_hbm.at[pl.ds(off, GC)])
        return carry
    lax.fori_loop(0, RPW // GC, chunk_body, 0)

    pltpu.sync_copy(sw_sh.at[pl.ds(base, RPW)], wslice_v)
    pltpu.sync_copy(wslice_v, ws_hbm.at[pl.ds(base, RPW)])


def _moe_ffn_body(eid_ref, xs_ref, ws_ref, g_ref, u_ref, d_ref, ys_ref):
    xb = xs_ref[...].astype(jnp.bfloat16)
    acc = jnp.zeros((BT, D), jnp.float32)
    for f in range(FF // FC):
        gw = g_ref[0, pl.ds(f * FC, FC), :]
        uw = u_ref[0, pl.ds(f * FC, FC), :]
        g = jax.lax.dot_general(
            xb, gw, (((1,), (1,)), ((), ())),
            preferred_element_type=jnp.float32)
        u = jax.lax.dot_general(
            xb, uw, (((1,), (1,)), ((), ())),
            preferred_element_type=jnp.float32)
        h = ((g * jax.nn.sigmoid(g)) * u).astype(jnp.bfloat16)
        dw = d_ref[0, :, pl.ds(f * FC, FC)]
        acc = acc + jax.lax.dot_general(
            h, dw, (((1,), (1,)), ((), ())),
            preferred_element_type=jnp.float32)
    ys_ref[...] = ws_ref[...] * acc


def _sc_combine_body(ys_hbm, dest_hbm, out_hbm,
                     idxa_v, idxb_v, bufa_v, bufb_v, sem):
    cid = lax.axis_index("c")
    sid = lax.axis_index("s")
    wid = cid * NS + sid

    def chunk_body(c, carry):
        t0 = wid * TPW + c * CT
        pltpu.sync_copy(dest_hbm.at[pl.ds(t0, CT)], idxa_v)
        pltpu.sync_copy(dest_hbm.at[pl.ds(T + t0, CT)], idxb_v)
        pltpu.async_copy(ys_hbm.at[idxa_v], bufa_v, sem).wait()
        pltpu.async_copy(ys_hbm.at[idxb_v], bufb_v, sem).wait()

        def add_row(r, cc):
            for k4 in range(D // 64):
                for j in range(4):
                    sl = pl.ds(k4 * 64 + j * 16, 16)
                    bufa_v[r, sl] = bufa_v[r, sl] + bufb_v[r, sl]
            return cc
        lax.fori_loop(0, CT, add_row, 0)
        pltpu.sync_copy(bufa_v, out_hbm.at[pl.ds(t0, CT)])
        return carry
    lax.fori_loop(0, TPW // CT, chunk_body, 0)


def kernel(x, Wr, gate, up, down):
    b, s, d = x.shape
    xf = x.reshape(T, D)

    bt_r = 1024
    logits, i12, w12 = pl.pallas_call(
        _router_body,
        grid=(T // bt_r,),
        in_specs=[
            pl.BlockSpec((bt_r, D), lambda i: (i, 0)),
            pl.BlockSpec((E, D), lambda i: (0, 0)),
        ],
        out_specs=[
            pl.BlockSpec((bt_r, E), lambda i: (i, 0)),
            pl.BlockSpec((bt_r, KTOP), lambda i: (i, 0)),
            pl.BlockSpec((bt_r, KTOP), lambda i: (i, 0)),
        ],
        out_shape=[
            jax.ShapeDtypeStruct((T, E), jnp.float32),
            jax.ShapeDtypeStruct((T, KTOP), jnp.int32),
            jax.ShapeDtypeStruct((T, KTOP), jnp.float32),
        ],
    )(xf, Wr)

    ti_sq = i12.T.reshape(SQ, SQ)
    tw_flat = w12.T.reshape(A)

    dest_sq, eid_pad = pl.pallas_call(
        _dispatch_body,
        grid=(1,),
        in_specs=[pl.BlockSpec((SQ, SQ), lambda i: (0, 0))],
        out_specs=[
            pl.BlockSpec((SQ, SQ), lambda i: (0, 0)),
            pl.BlockSpec((1, SQ), lambda i: (0, 0)),
        ],
        out_shape=[
            jax.ShapeDtypeStruct((SQ, SQ), jnp.int32),
            jax.ShapeDtypeStruct((1, SQ), jnp.int32),
        ],
    )(ti_sq)
    dest_flat = dest_sq.reshape(A)
    tile_eid = eid_pad.reshape(SQ)[:NL]

    sc_mesh = plsc.VectorSubcoreMesh(core_axis_name="c", subcore_axis_name="s", num_cores=NC, num_subcores=NS)
    xs, ws = pl.kernel(
        _sc_dispatch_body,
        out_type=[
            jax.ShapeDtypeStruct((L, D), jnp.float32),
            jax.ShapeDtypeStruct((L,), jnp.float32),
        ],
        mesh=sc_mesh,
        scratch_types=[
            pltpu.VMEM((L,), jnp.int32),
            pltpu.VMEM((L,), jnp.float32),
            pltpu.VMEM((SQ, SQ), jnp.int32),
            pltpu.VMEM((A,), jnp.float32),
            pltpu.VMEM((A,), jnp.int32),
            pltpu.VMEM((GC,), jnp.int32),
            pltpu.VMEM((GC, D), jnp.float32),
            pltpu.VMEM((RPW,), jnp.float32),
            pltpu.VMEM_SHARED((L,), jnp.int32),
            pltpu.VMEM_SHARED((L,), jnp.float32),
            pltpu.SemaphoreType.DMA,
        ],
    )(dest_sq, tw_flat, xf)

    gate_b = gate.astype(jnp.bfloat16)
    up_b = up.astype(jnp.bfloat16)
    down_b = down.astype(jnp.bfloat16)
    ws2 = ws.reshape(L, 1)

    grid_spec = pltpu.PrefetchScalarGridSpec(
        num_scalar_prefetch=1,
        grid=(NL,),
        in_specs=[
            pl.BlockSpec((BT, D), lambda i, eid: (i, 0)),
            pl.BlockSpec((BT, 1), lambda i, eid: (i, 0)),
            pl.BlockSpec((1, FF, D), lambda i, eid: (eid[i], 0, 0)),
            pl.BlockSpec((1, FF, D), lambda i, eid: (eid[i], 0, 0)),
            pl.BlockSpec((1, D, FF), lambda i, eid: (eid[i], 0, 0)),
        ],
        out_specs=pl.BlockSpec((BT, D), lambda i, eid: (i, 0)),
    )
    ys = pl.pallas_call(
        _moe_ffn_body,
        grid_spec=grid_spec,
        out_shape=jax.ShapeDtypeStruct((L, D), jnp.float32),
        compiler_params=pltpu.CompilerParams(
            dimension_semantics=("arbitrary",)),
    )(tile_eid, xs, ws2, gate_b, up_b, down_b)

    final = pl.kernel(
        _sc_combine_body,
        out_type=jax.ShapeDtypeStruct((T, D), jnp.float32),
        mesh=sc_mesh,
        scratch_types=[
            pltpu.VMEM((CT,), jnp.int32),
            pltpu.VMEM((CT,), jnp.int32),
            pltpu.VMEM((CT, D), jnp.float32),
            pltpu.VMEM((CT, D), jnp.float32),
            pltpu.SemaphoreType.DMA,
        ],
    )(ys, dest_flat)

    return final.reshape(b, s, d), logits


# speedup vs baseline: 2.2990x; 1.0090x over previous
import functools

import jax
import jax.numpy as jnp
from jax import lax
from jax.experimental import pallas as pl
from jax.experimental.pallas import tpu as pltpu
from jax.experimental.pallas import tpu_sc as plsc

T = 8192
D = 1024
E = 8
FF = 4096
KTOP = 2
A = T * KTOP
SQ = 128
BT = 256
L = A + E * BT
NL = L // BT
FC = 512

NC = 2
NS = 16
NW = NC * NS
RPW = L // NW
GC = 48
TPW = T // NW
CT = 32


def _router_body(x_ref, wr_ref, logits_ref, i12_ref, w12_ref):
    x = x_ref[...]
    wr = wr_ref[...]
    logits = jax.lax.dot_general(
        x, wr, (((1,), (1,)), ((), ())), preferred_element_type=jnp.float32)
    logits_ref[...] = logits
    m = jnp.max(logits, axis=1, keepdims=True)
    unnorm = jnp.exp(logits - m)
    p = unnorm / jnp.sum(unnorm, axis=1, keepdims=True)
    cols = jax.lax.broadcasted_iota(jnp.int32, p.shape, 1)
    i1 = jnp.argmax(p, axis=1)[:, None]
    p_m = jnp.where(cols == i1, -jnp.inf, p)
    i2 = jnp.argmax(p_m, axis=1)[:, None]
    w1 = jnp.sum(jnp.where(cols == i1, p, 0.0), axis=1, keepdims=True)
    w2 = jnp.sum(jnp.where(cols == i2, p, 0.0), axis=1, keepdims=True)
    i12_ref[...] = jnp.concatenate([i1, i2], axis=1)
    w12_ref[...] = jnp.concatenate([w1, w2], axis=1)


def _dispatch_body(ti_ref, dest_ref, eid_ref):
    ti = ti_ref[...]
    r_i = jax.lax.broadcasted_iota(jnp.int32, (SQ, SQ), 0)
    c_i = jax.lax.broadcasted_iota(jnp.int32, (SQ, SQ), 1)
    incl = (r_i <= c_i).astype(jnp.float32)
    strict = (c_i < r_i).astype(jnp.float32)

    masks, counts = [], []
    ranks = []
    for e in range(E):
        mf = (ti == e).astype(jnp.float32)
        cum = jax.lax.dot_general(
            mf, incl, (((1,), (0,)), ((), ())),
            preferred_element_type=jnp.float32)
        rowsum = cum[:, SQ - 1:SQ]
        rowpref = jax.lax.dot_general(
            strict, rowsum, (((1,), (0,)), ((), ())),
            preferred_element_type=jnp.float32)
        rank = cum - mf + rowpref
        masks.append(mf)
        ranks.append(rank)
        counts.append(jnp.sum(mf))

    base = 0.0
    dest = jnp.zeros((SQ, SQ), jnp.float32)
    bases = []
    for e in range(E):
        bases.append(base)
        dest = dest + masks[e] * (ranks[e] + base)
        padded = jnp.ceil(counts[e] / BT) * BT
        base = base + padded
    dest_ref[...] = dest.astype(jnp.int32)

    ts = jax.lax.broadcasted_iota(jnp.int32, (1, SQ), 1).astype(
        jnp.float32) * BT
    eid = jnp.zeros((1, SQ), jnp.int32)
    for e in range(1, E):
        eid = eid + (ts >= bases[e]).astype(jnp.int32)
    eid_ref[...] = eid


def _sc_dispatch_body(dest_hbm, tw_hbm, x_hbm, xs_hbm, ws_hbm,
                      zt_v, zw_v, dst8_v, tw_v, vals_v,
                      idx_a, idx_b, rows_a, rows_b, wslice_v,
                      stok_sh, sw_sh, sem):
    cid = lax.axis_index("c")
    sid = lax.axis_index("s")

    zs = L // NS
    sqr = SQ // NS
    ac = sqr * SQ

    def zb(i, c):
        zt_v[pl.ds(i * 16, 16)] = jnp.zeros((16,), jnp.int32)
        zw_v[pl.ds(i * 16, 16)] = jnp.zeros((16,), jnp.float32)
        return c
    lax.fori_loop(0, zs // 16, zb, 0)
    pltpu.sync_copy(zt_v, stok_sh.at[pl.ds(sid * zs, zs)])
    pltpu.sync_copy(zw_v, sw_sh.at[pl.ds(sid * zs, zs)])

    pltpu.sync_copy(dest_hbm.at[pl.ds(sid * sqr, sqr)], dst8_v)
    pltpu.sync_copy(tw_hbm.at[pl.ds(sid * ac, ac)], tw_v)

    def vb(i, c):
        va = sid * ac + i * 16 + jax.lax.iota(jnp.int32, 16)
        vals_v[pl.ds(i * 16, 16)] = jnp.bitwise_and(va, T - 1)
        return c
    lax.fori_loop(0, ac // 16, vb, 0)

    plsc.subcore_barrier()

    hs = []
    for j in range(sqr):
        hs.append(pltpu.async_copy(
            vals_v.at[pl.ds(j * SQ, SQ)], stok_sh.at[dst8_v.at[j]], sem))
        hs.append(pltpu.async_copy(
            tw_v.at[pl.ds(j * SQ, SQ)], sw_sh.at[dst8_v.at[j]], sem))
    for h in hs:
        h.wait()

    plsc.subcore_barrier()

    wid = cid * NS + sid
    base = wid * RPW
    nch = RPW // GC
    bufs = ((idx_a, rows_a), (idx_b, rows_b))

    def start(c):
        idx, rows = bufs[c % 2]
        pltpu.sync_copy(stok_sh.at[pl.ds(base + c * GC, GC)], idx)
        return pltpu.async_copy(x_hbm.at[idx], rows, sem)

    pending = start(0)
    for c in range(nch):
        nxt = start(c + 1) if c + 1 < nch else None
        pending.wait()
        pltpu.sync_copy(bufs[c % 2][1], xs_hbm.at[pl.ds(base + c * GC, GC)])
        pending = nxt

    pltpu.sync_copy(sw_sh.at[pl.ds(base, RPW)], wslice_v)
    pltpu.sync_copy(wslice_v, ws_hbm.at[pl.ds(base, RPW)])


def _moe_ffn_body(eid_ref, xs_ref, ws_ref, g_ref, u_ref, d_ref, ys_ref):
    xb = xs_ref[...].astype(jnp.bfloat16)
    acc = jnp.zeros((BT, D), jnp.float32)
    for f in range(FF // FC):
        gw = g_ref[0, pl.ds(f * FC, FC), :]
        uw = u_ref[0, pl.ds(f * FC, FC), :]
        g = jax.lax.dot_general(
            xb, gw, (((1,), (1,)), ((), ())),
            preferred_element_type=jnp.float32)
        u = jax.lax.dot_general(
            xb, uw, (((1,), (1,)), ((), ())),
            preferred_element_type=jnp.float32)
        h = ((g * jax.nn.sigmoid(g)) * u).astype(jnp.bfloat16)
        dw = d_ref[0, :, pl.ds(f * FC, FC)]
        acc = acc + jax.lax.dot_general(
            h, dw, (((1,), (1,)), ((), ())),
            preferred_element_type=jnp.float32)
    ys_ref[...] = ws_ref[...] * acc


def _sc_combine_body(ys_hbm, dest_hbm, out_hbm,
                     idxa_v, idxb_v, bufa_v, bufb_v, sem):
    cid = lax.axis_index("c")
    sid = lax.axis_index("s")
    wid = cid * NS + sid

    def chunk_body(c, carry):
        t0 = wid * TPW + c * CT
        pltpu.sync_copy(dest_hbm.at[pl.ds(t0, CT)], idxa_v)
        pltpu.sync_copy(dest_hbm.at[pl.ds(T + t0, CT)], idxb_v)
        pltpu.async_copy(ys_hbm.at[idxa_v], bufa_v, sem).wait()
        pltpu.async_copy(ys_hbm.at[idxb_v], bufb_v, sem).wait()

        def add_row(r, cc):
            for k4 in range(D // 64):
                for j in range(4):
                    sl = pl.ds(k4 * 64 + j * 16, 16)
                    bufa_v[r, sl] = bufa_v[r, sl] + bufb_v[r, sl]
            return cc
        lax.fori_loop(0, CT, add_row, 0)
        pltpu.sync_copy(bufa_v, out_hbm.at[pl.ds(t0, CT)])
        return carry
    lax.fori_loop(0, TPW // CT, chunk_body, 0)


def kernel(x, Wr, gate, up, down):
    b, s, d = x.shape
    xf = x.reshape(T, D)

    bt_r = 1024
    logits, i12, w12 = pl.pallas_call(
        _router_body,
        grid=(T // bt_r,),
        in_specs=[
            pl.BlockSpec((bt_r, D), lambda i: (i, 0)),
            pl.BlockSpec((E, D), lambda i: (0, 0)),
        ],
        out_specs=[
            pl.BlockSpec((bt_r, E), lambda i: (i, 0)),
            pl.BlockSpec((bt_r, KTOP), lambda i: (i, 0)),
            pl.BlockSpec((bt_r, KTOP), lambda i: (i, 0)),
        ],
        out_shape=[
            jax.ShapeDtypeStruct((T, E), jnp.float32),
            jax.ShapeDtypeStruct((T, KTOP), jnp.int32),
            jax.ShapeDtypeStruct((T, KTOP), jnp.float32),
        ],
    )(xf, Wr)

    ti_sq = i12.T.reshape(SQ, SQ)
    tw_flat = w12.T.reshape(A)

    dest_sq, eid_pad = pl.pallas_call(
        _dispatch_body,
        grid=(1,),
        in_specs=[pl.BlockSpec((SQ, SQ), lambda i: (0, 0))],
        out_specs=[
            pl.BlockSpec((SQ, SQ), lambda i: (0, 0)),
            pl.BlockSpec((1, SQ), lambda i: (0, 0)),
        ],
        out_shape=[
            jax.ShapeDtypeStruct((SQ, SQ), jnp.int32),
            jax.ShapeDtypeStruct((1, SQ), jnp.int32),
        ],
    )(ti_sq)
    dest_flat = dest_sq.reshape(A)
    tile_eid = eid_pad.reshape(SQ)[:NL]

    sc_mesh = plsc.VectorSubcoreMesh(core_axis_name="c", subcore_axis_name="s", num_cores=NC, num_subcores=NS)
    xs, ws = pl.kernel(
        _sc_dispatch_body,
        out_type=[
            jax.ShapeDtypeStruct((L, D), jnp.float32),
            jax.ShapeDtypeStruct((L,), jnp.float32),
        ],
        mesh=sc_mesh,
        scratch_types=[
            pltpu.VMEM((L // NS,), jnp.int32),
            pltpu.VMEM((L // NS,), jnp.float32),
            pltpu.VMEM((SQ // NS, SQ), jnp.int32),
            pltpu.VMEM((SQ // NS * SQ,), jnp.float32),
            pltpu.VMEM((SQ // NS * SQ,), jnp.int32),
            pltpu.VMEM((GC,), jnp.int32),
            pltpu.VMEM((GC,), jnp.int32),
            pltpu.VMEM((GC, D), jnp.float32),
            pltpu.VMEM((GC, D), jnp.float32),
            pltpu.VMEM((RPW,), jnp.float32),
            pltpu.VMEM_SHARED((L,), jnp.int32),
            pltpu.VMEM_SHARED((L,), jnp.float32),
            pltpu.SemaphoreType.DMA,
        ],
    )(dest_sq, tw_flat, xf)

    gate_b = gate.astype(jnp.bfloat16)
    up_b = up.astype(jnp.bfloat16)
    down_b = down.astype(jnp.bfloat16)
    ws2 = ws.reshape(L, 1)

    grid_spec = pltpu.PrefetchScalarGridSpec(
        num_scalar_prefetch=1,
        grid=(NL,),
        in_specs=[
            pl.BlockSpec((BT, D), lambda i, eid: (i, 0)),
            pl.BlockSpec((BT, 1), lambda i, eid: (i, 0)),
            pl.BlockSpec((1, FF, D), lambda i, eid: (eid[i], 0, 0)),
            pl.BlockSpec((1, FF, D), lambda i, eid: (eid[i], 0, 0)),
            pl.BlockSpec((1, D, FF), lambda i, eid: (eid[i], 0, 0)),
        ],
        out_specs=pl.BlockSpec((BT, D), lambda i, eid: (i, 0)),
    )
    ys = pl.pallas_call(
        _moe_ffn_body,
        grid_spec=grid_spec,
        out_shape=jax.ShapeDtypeStruct((L, D), jnp.float32),
        compiler_params=pltpu.CompilerParams(
            dimension_semantics=("arbitrary",)),
    )(tile_eid, xs, ws2, gate_b, up_b, down_b)

    final = pl.kernel(
        _sc_combine_body,
        out_type=jax.ShapeDtypeStruct((T, D), jnp.float32),
        mesh=sc_mesh,
        scratch_types=[
            pltpu.VMEM((CT,), jnp.int32),
            pltpu.VMEM((CT,), jnp.int32),
            pltpu.VMEM((CT, D), jnp.float32),
            pltpu.VMEM((CT, D), jnp.float32),
            pltpu.SemaphoreType.DMA,
        ],
    )(ys, dest_flat)

    return final.reshape(b, s, d), logits


# speedup vs baseline: 2.3851x; 1.0375x over previous
import functools

import jax
import jax.numpy as jnp
from jax import lax
from jax.experimental import pallas as pl
from jax.experimental.pallas import tpu as pltpu
from jax.experimental.pallas import tpu_sc as plsc

T = 8192
D = 1024
E = 8
FF = 4096
KTOP = 2
A = T * KTOP
SQ = 128
BT = 256
L = A + E * BT
NL = L // BT
FC = 1024

NC = 2
NS = 16
NW = NC * NS
RPW = L // NW
GC = 48
TPW = T // NW
CT = 32


def _router_body(x_ref, wr_ref, logits_ref, i12_ref, w12_ref):
    x = x_ref[...]
    wr = wr_ref[...]
    logits = jax.lax.dot_general(
        x, wr, (((1,), (1,)), ((), ())), preferred_element_type=jnp.float32)
    logits_ref[...] = logits
    m = jnp.max(logits, axis=1, keepdims=True)
    unnorm = jnp.exp(logits - m)
    p = unnorm / jnp.sum(unnorm, axis=1, keepdims=True)
    cols = jax.lax.broadcasted_iota(jnp.int32, p.shape, 1)
    i1 = jnp.argmax(p, axis=1)[:, None]
    p_m = jnp.where(cols == i1, -jnp.inf, p)
    i2 = jnp.argmax(p_m, axis=1)[:, None]
    w1 = jnp.sum(jnp.where(cols == i1, p, 0.0), axis=1, keepdims=True)
    w2 = jnp.sum(jnp.where(cols == i2, p, 0.0), axis=1, keepdims=True)
    i12_ref[...] = jnp.concatenate([i1, i2], axis=1)
    w12_ref[...] = jnp.concatenate([w1, w2], axis=1)


def _dispatch_body(ti_ref, dest_ref, eid_ref):
    ti = ti_ref[...]
    r_i = jax.lax.broadcasted_iota(jnp.int32, (SQ, SQ), 0)
    c_i = jax.lax.broadcasted_iota(jnp.int32, (SQ, SQ), 1)
    incl = (r_i <= c_i).astype(jnp.float32)
    strict = (c_i < r_i).astype(jnp.float32)

    masks, counts = [], []
    ranks = []
    for e in range(E):
        mf = (ti == e).astype(jnp.float32)
        cum = jax.lax.dot_general(
            mf, incl, (((1,), (0,)), ((), ())),
            preferred_element_type=jnp.float32)
        rowsum = cum[:, SQ - 1:SQ]
        rowpref = jax.lax.dot_general(
            strict, rowsum, (((1,), (0,)), ((), ())),
            preferred_element_type=jnp.float32)
        rank = cum - mf + rowpref
        masks.append(mf)
        ranks.append(rank)
        counts.append(jnp.sum(mf))

    base = 0.0
    dest = jnp.zeros((SQ, SQ), jnp.float32)
    bases = []
    for e in range(E):
        bases.append(base)
        dest = dest + masks[e] * (ranks[e] + base)
        padded = jnp.ceil(counts[e] / BT) * BT
        base = base + padded
    dest_ref[...] = dest.astype(jnp.int32)

    ts = jax.lax.broadcasted_iota(jnp.int32, (1, SQ), 1).astype(
        jnp.float32) * BT
    eid = jnp.zeros((1, SQ), jnp.int32)
    for e in range(1, E):
        eid = eid + (ts >= bases[e]).astype(jnp.int32)
    eid_ref[...] = eid


def _sc_dispatch_body(dest_hbm, tw_hbm, x_hbm, xs_hbm, ws_hbm,
                      zt_v, zw_v, dst8_v, tw_v, vals_v,
                      idxall_v, rows_a, rows_b, wslice_v,
                      stok_sh, sw_sh, sem, sem2):
    cid = lax.axis_index("c")
    sid = lax.axis_index("s")

    zs = L // NS
    sqr = SQ // NS
    ac = sqr * SQ

    def zb(i, c):
        zt_v[pl.ds(i * 16, 16)] = jnp.zeros((16,), jnp.int32)
        zw_v[pl.ds(i * 16, 16)] = jnp.zeros((16,), jnp.float32)
        return c
    lax.fori_loop(0, zs // 16, zb, 0)
    pltpu.sync_copy(zt_v, stok_sh.at[pl.ds(sid * zs, zs)])
    pltpu.sync_copy(zw_v, sw_sh.at[pl.ds(sid * zs, zs)])

    pltpu.sync_copy(dest_hbm.at[pl.ds(sid * sqr, sqr)], dst8_v)
    pltpu.sync_copy(tw_hbm.at[pl.ds(sid * ac, ac)], tw_v)

    def vb(i, c):
        va = sid * ac + i * 16 + jax.lax.iota(jnp.int32, 16)
        vals_v[pl.ds(i * 16, 16)] = jnp.bitwise_and(va, T - 1)
        return c
    lax.fori_loop(0, ac // 16, vb, 0)

    plsc.subcore_barrier()

    hs = []
    for j in range(sqr):
        hs.append(pltpu.async_copy(
            vals_v.at[pl.ds(j * SQ, SQ)], stok_sh.at[dst8_v.at[j]], sem))
        hs.append(pltpu.async_copy(
            tw_v.at[pl.ds(j * SQ, SQ)], sw_sh.at[dst8_v.at[j]], sem))
    for h in hs:
        h.wait()

    plsc.subcore_barrier()

    wid = cid * NS + sid
    base = wid * RPW
    nch = RPW // GC
    rowbufs = (rows_a, rows_b)

    pltpu.sync_copy(stok_sh.at[pl.ds(base, RPW)], idxall_v)

    def start(c):
        return pltpu.async_copy(
            x_hbm.at[idxall_v.at[pl.ds(c * GC, GC)]], rowbufs[c % 2], sem)

    pend_g = start(0)
    pend_w = [None, None]
    for c in range(nch):
        if c + 1 < nch:
            b2 = (c + 1) % 2
            if pend_w[b2] is not None:
                pend_w[b2].wait()
            nxt = start(c + 1)
        else:
            nxt = None
        pend_g.wait()
        pend_w[c % 2] = pltpu.async_copy(
            rowbufs[c % 2], xs_hbm.at[pl.ds(base + c * GC, GC)], sem2)
        pend_g = nxt
    for h in pend_w:
        if h is not None:
            h.wait()

    pltpu.sync_copy(sw_sh.at[pl.ds(base, RPW)], wslice_v)
    pltpu.sync_copy(wslice_v, ws_hbm.at[pl.ds(base, RPW)])


def _moe_ffn_body(eid_ref, xs_ref, ws_ref, g_ref, u_ref, d_ref, ys_ref):
    xb = xs_ref[...].astype(jnp.bfloat16)
    acc = jnp.zeros((BT, D), jnp.float32)
    for f in range(FF // FC):
        gw = g_ref[0, pl.ds(f * FC, FC), :]
        uw = u_ref[0, pl.ds(f * FC, FC), :]
        g = jax.lax.dot_general(
            xb, gw, (((1,), (1,)), ((), ())),
            preferred_element_type=jnp.float32)
        u = jax.lax.dot_general(
            xb, uw, (((1,), (1,)), ((), ())),
            preferred_element_type=jnp.float32)
        h = ((g * jax.nn.sigmoid(g)) * u).astype(jnp.bfloat16)
        dw = d_ref[0, :, pl.ds(f * FC, FC)]
        acc = acc + jax.lax.dot_general(
            h, dw, (((1,), (1,)), ((), ())),
            preferred_element_type=jnp.float32)
    ys_ref[...] = ws_ref[...] * acc


def _sc_combine_body(ys_hbm, dest_hbm, out_hbm,
                     idxa_v, idxb_v, bufa_v, bufb_v, sem):
    cid = lax.axis_index("c")
    sid = lax.axis_index("s")
    wid = cid * NS + sid

    def chunk_body(c, carry):
        t0 = wid * TPW + c * CT
        pltpu.sync_copy(dest_hbm.at[pl.ds(t0, CT)], idxa_v)
        pltpu.sync_copy(dest_hbm.at[pl.ds(T + t0, CT)], idxb_v)
        pltpu.async_copy(ys_hbm.at[idxa_v], bufa_v, sem).wait()
        pltpu.async_copy(ys_hbm.at[idxb_v], bufb_v, sem).wait()

        def add_row(r, cc):
            for k4 in range(D // 64):
                for j in range(4):
                    sl = pl.ds(k4 * 64 + j * 16, 16)
                    bufa_v[r, sl] = bufa_v[r, sl] + bufb_v[r, sl]
            return cc
        lax.fori_loop(0, CT, add_row, 0)
        pltpu.sync_copy(bufa_v, out_hbm.at[pl.ds(t0, CT)])
        return carry
    lax.fori_loop(0, TPW // CT, chunk_body, 0)


def kernel(x, Wr, gate, up, down):
    b, s, d = x.shape
    xf = x.reshape(T, D)

    bt_r = 1024
    logits, i12, w12 = pl.pallas_call(
        _router_body,
        grid=(T // bt_r,),
        in_specs=[
            pl.BlockSpec((bt_r, D), lambda i: (i, 0)),
            pl.BlockSpec((E, D), lambda i: (0, 0)),
        ],
        out_specs=[
            pl.BlockSpec((bt_r, E), lambda i: (i, 0)),
            pl.BlockSpec((bt_r, KTOP), lambda i: (i, 0)),
            pl.BlockSpec((bt_r, KTOP), lambda i: (i, 0)),
        ],
        out_shape=[
            jax.ShapeDtypeStruct((T, E), jnp.float32),
            jax.ShapeDtypeStruct((T, KTOP), jnp.int32),
            jax.ShapeDtypeStruct((T, KTOP), jnp.float32),
        ],
    )(xf, Wr)

    ti_sq = i12.T.reshape(SQ, SQ)
    tw_flat = w12.T.reshape(A)

    dest_sq, eid_pad = pl.pallas_call(
        _dispatch_body,
        grid=(1,),
        in_specs=[pl.BlockSpec((SQ, SQ), lambda i: (0, 0))],
        out_specs=[
            pl.BlockSpec((SQ, SQ), lambda i: (0, 0)),
            pl.BlockSpec((1, SQ), lambda i: (0, 0)),
        ],
        out_shape=[
            jax.ShapeDtypeStruct((SQ, SQ), jnp.int32),
            jax.ShapeDtypeStruct((1, SQ), jnp.int32),
        ],
    )(ti_sq)
    dest_flat = dest_sq.reshape(A)
    tile_eid = eid_pad.reshape(SQ)[:NL]

    sc_mesh = plsc.VectorSubcoreMesh(core_axis_name="c", subcore_axis_name="s", num_cores=NC, num_subcores=NS)
    xs, ws = pl.kernel(
        _sc_dispatch_body,
        out_type=[
            jax.ShapeDtypeStruct((L, D), jnp.float32),
            jax.ShapeDtypeStruct((L,), jnp.float32),
        ],
        mesh=sc_mesh,
        scratch_types=[
            pltpu.VMEM((L // NS,), jnp.int32),
            pltpu.VMEM((L // NS,), jnp.float32),
            pltpu.VMEM((SQ // NS, SQ), jnp.int32),
            pltpu.VMEM((SQ // NS * SQ,), jnp.float32),
            pltpu.VMEM((SQ // NS * SQ,), jnp.int32),
            pltpu.VMEM((RPW,), jnp.int32),
            pltpu.VMEM((GC, D), jnp.float32),
            pltpu.VMEM((GC, D), jnp.float32),
            pltpu.VMEM((RPW,), jnp.float32),
            pltpu.VMEM_SHARED((L,), jnp.int32),
            pltpu.VMEM_SHARED((L,), jnp.float32),
            pltpu.SemaphoreType.DMA,
            pltpu.SemaphoreType.DMA,
        ],
    )(dest_sq, tw_flat, xf)

    gate_b = gate.astype(jnp.bfloat16)
    up_b = up.astype(jnp.bfloat16)
    down_b = down.astype(jnp.bfloat16)
    ws2 = ws.reshape(L, 1)

    grid_spec = pltpu.PrefetchScalarGridSpec(
        num_scalar_prefetch=1,
        grid=(NL,),
        in_specs=[
            pl.BlockSpec((BT, D), lambda i, eid: (i, 0)),
            pl.BlockSpec((BT, 1), lambda i, eid: (i, 0)),
            pl.BlockSpec((1, FF, D), lambda i, eid: (eid[i], 0, 0)),
            pl.BlockSpec((1, FF, D), lambda i, eid: (eid[i], 0, 0)),
            pl.BlockSpec((1, D, FF), lambda i, eid: (eid[i], 0, 0)),
        ],
        out_specs=pl.BlockSpec((BT, D), lambda i, eid: (i, 0)),
    )
    ys = pl.pallas_call(
        _moe_ffn_body,
        grid_spec=grid_spec,
        out_shape=jax.ShapeDtypeStruct((L, D), jnp.float32),
        compiler_params=pltpu.CompilerParams(
            dimension_semantics=("arbitrary",)),
    )(tile_eid, xs, ws2, gate_b, up_b, down_b)

    final = pl.kernel(
        _sc_combine_body,
        out_type=jax.ShapeDtypeStruct((T, D), jnp.float32),
        mesh=sc_mesh,
        scratch_types=[
            pltpu.VMEM((CT,), jnp.int32),
            pltpu.VMEM((CT,), jnp.int32),
            pltpu.VMEM((CT, D), jnp.float32),
            pltpu.VMEM((CT, D), jnp.float32),
            pltpu.SemaphoreType.DMA,
        ],
    )(ys, dest_flat)

    return final.reshape(b, s, d), logits
